# baseline (speedup 1.0000x reference)
import functools

import jax
import jax.numpy as jnp
from jax import lax
from jax.experimental import pallas as pl
from jax.experimental.pallas import tpu as pltpu
from jax.experimental.pallas import tpu_sc as plsc

M_ROWS = 1_000_000
C = 128
B = 16384

NC = 2
NS = 16
L = 16
NW = NC * NS
B_PER_W = B // NW
CHUNK = 128
N_CHUNKS = B_PER_W // CHUNK
C_VECS = C // L

P_C0 = 0.00014158017492754693
P_C1 = 0.995426661775425
P_C2 = -0.4640707011025748
P_C3 = 0.21640858368174304
P_C4 = -0.05486231128931281

_mesh = plsc.VectorSubcoreMesh(
    core_axis_name="c", subcore_axis_name="s", num_cores=NC, num_subcores=NS
)


@functools.partial(
    pl.kernel,
    out_type=jax.ShapeDtypeStruct((NW, L), jnp.float32),
    mesh=_mesh,
    scratch_types=[
        pltpu.VMEM((N_CHUNKS, CHUNK), jnp.int32),
        pltpu.VMEM((2, CHUNK, C), jnp.float32),
        pltpu.VMEM((2, CHUNK, C), jnp.float32),
        pltpu.VMEM((L,), jnp.float32),
        pltpu.SemaphoreType.DMA,
        pltpu.SemaphoreType.DMA,
        pltpu.SemaphoreType.DMA,
        pltpu.SemaphoreType.DMA,
    ],
)
def _sc_loss(outputs_hbm, idx_hbm, conf_hbm, out_hbm,
             idx_v, rows_v, outs_v, acc_v,
             gsem0, gsem1, osem0, osem1):
    wid = lax.axis_index("s") * NC + lax.axis_index("c")
    base = wid * B_PER_W
    gsems = (gsem0, gsem1)
    osems = (osem0, osem1)

    def start(g):
        slot = g % 2
        pltpu.sync_copy(idx_hbm.at[pl.ds(base + g * CHUNK, CHUNK)], idx_v.at[g])
        gather = pltpu.async_copy(conf_hbm.at[idx_v.at[g]], rows_v.at[slot],
                                  gsems[slot])
        ocopy = pltpu.async_copy(
            outputs_hbm.at[pl.ds(base + g * CHUNK, CHUNK)], outs_v.at[slot],
            osems[slot])
        return gather, ocopy

    pending = {0: start(0), 1: start(1)}

    accs = tuple(jnp.zeros((L,), jnp.float32) for _ in range(C_VECS))
    for g in range(N_CHUNKS):
        slot = g % 2
        gather, ocopy = pending.pop(g)
        gather.wait()
        ocopy.wait()

        def body(r, accs, slot=slot):
            new = []
            for cc in range(C_VECS):
                x = outs_v[slot, r, pl.ds(cc * L, L)]
                t = rows_v[slot, r, pl.ds(cc * L, L)]
                y = jnp.exp(jnp.minimum(x, -x))
                p = (((P_C4 * y + P_C3) * y + P_C2) * y + P_C1) * y
                new.append(accs[cc] + (jnp.maximum(x, 0.0) - x * t + p))
            return tuple(new)
        accs = lax.fori_loop(0, CHUNK, body, accs)
        if g + 2 < N_CHUNKS:
            pending[g + 2] = start(g + 2)

    total = accs[0]
    for cc in range(1, C_VECS):
        total = total + accs[cc]
    total = total + jnp.float32(P_C0 * B_PER_W * C_VECS)
    acc_v[...] = total
    pltpu.sync_copy(acc_v, out_hbm.at[wid])


def _tc_combine_body(part_ref, out_ref):
    out_ref[0, 0] = jnp.sum(part_ref[...]) * (1.0 / (B * C))


_tc_combine = pl.pallas_call(
    _tc_combine_body,
    in_specs=[pl.BlockSpec((NW, L), lambda: (0, 0))],
    out_specs=pl.BlockSpec(memory_space=pltpu.SMEM),
    out_shape=jax.ShapeDtypeStruct((1, 1), jnp.float32),
)


def kernel(outputs, index, confidence):
    partials = _sc_loss(outputs, index, confidence)
    return _tc_combine(partials)[0, 0]


# speedup vs baseline: 1.1913x; 1.1913x over previous
import functools

import jax
import jax.numpy as jnp
from jax import lax
from jax.experimental import pallas as pl
from jax.experimental.pallas import tpu as pltpu
from jax.experimental.pallas import tpu_sc as plsc

M_ROWS = 1_000_000
C = 128
B = 16384

NC = 2
NS = 16
L = 16
NW = NC * NS
B_PER_W = B // NW
CHUNK = 128
N_CHUNKS = B_PER_W // CHUNK
NBUF = 3
C_VECS = C // L

_mesh = plsc.VectorSubcoreMesh(
    core_axis_name="c", subcore_axis_name="s", num_cores=NC, num_subcores=NS
)


@functools.partial(
    pl.kernel,
    out_type=jax.ShapeDtypeStruct((NW, L), jnp.float32),
    mesh=_mesh,
    scratch_types=[
        pltpu.VMEM((B_PER_W,), jnp.int32),
        pltpu.VMEM((NBUF, CHUNK, C), jnp.float32),
        pltpu.VMEM((NBUF, CHUNK, C), jnp.float32),
        pltpu.VMEM((L,), jnp.float32),
        pltpu.SemaphoreType.DMA,
        pltpu.SemaphoreType.DMA,
        pltpu.SemaphoreType.DMA,
        pltpu.SemaphoreType.DMA,
        pltpu.SemaphoreType.DMA,
        pltpu.SemaphoreType.DMA,
    ],
)
def _sc_gather_dot(outputs_hbm, idx_hbm, conf_hbm, out_hbm,
                   idx_v, rows_v, outs_v, acc_v, *sems):
    wid = lax.axis_index("s") * NC + lax.axis_index("c")
    base = wid * B_PER_W
    gsems = sems[:NBUF]
    osems = sems[NBUF:]

    pltpu.sync_copy(idx_hbm.at[pl.ds(base, B_PER_W)], idx_v)

    def start(g):
        slot = g % NBUF
        gather = pltpu.async_copy(
            conf_hbm.at[idx_v.at[pl.ds(g * CHUNK, CHUNK)]], rows_v.at[slot],
            gsems[slot])
        ocopy = pltpu.async_copy(
            outputs_hbm.at[pl.ds(base + g * CHUNK, CHUNK)], outs_v.at[slot],
            osems[slot])
        return gather, ocopy

    pending = {g: start(g) for g in range(NBUF)}

    accs = tuple(jnp.zeros((L,), jnp.float32) for _ in range(C_VECS))
    for g in range(N_CHUNKS):
        slot = g % NBUF
        gather, ocopy = pending.pop(g)
        gather.wait()
        ocopy.wait()

        def body(r, accs, slot=slot):
            return tuple(
                accs[cc]
                + rows_v[slot, r, pl.ds(cc * L, L)]
                * outs_v[slot, r, pl.ds(cc * L, L)]
                for cc in range(C_VECS)
            )
        accs = lax.fori_loop(0, CHUNK, body, accs)
        if g + NBUF < N_CHUNKS:
            pending[g + NBUF] = start(g + NBUF)

    total = accs[0]
    for cc in range(1, C_VECS):
        total = total + accs[cc]
    acc_v[...] = total
    pltpu.sync_copy(acc_v, out_hbm.at[wid])


TC_BLOCK = 2048
N_TC_BLOCKS = B // TC_BLOCK


def _tc_body(x_ref, out_ref):
    i = pl.program_id(0)
    x = x_ref[...]
    s = jnp.sum(jnp.maximum(x, 0.0) + jnp.log1p(jnp.exp(-jnp.abs(x))))

    @pl.when(i == 0)
    def _init():
        out_ref[0, 0] = 0.0

    out_ref[0, 0] += s


_tc_dense = pl.pallas_call(
    _tc_body,
    grid=(N_TC_BLOCKS,),
    in_specs=[pl.BlockSpec((TC_BLOCK, C), lambda i: (i, 0))],
    out_specs=pl.BlockSpec(memory_space=pltpu.SMEM),
    out_shape=jax.ShapeDtypeStruct((1, 1), jnp.float32),
)


def _tc_combine_body(dense_ref, part_ref, out_ref):
    out_ref[0, 0] = (dense_ref[0, 0] - jnp.sum(part_ref[...])) * (1.0 / (B * C))


_tc_combine = pl.pallas_call(
    _tc_combine_body,
    in_specs=[
        pl.BlockSpec(memory_space=pltpu.SMEM),
        pl.BlockSpec((NW, L), lambda: (0, 0)),
    ],
    out_specs=pl.BlockSpec(memory_space=pltpu.SMEM),
    out_shape=jax.ShapeDtypeStruct((1, 1), jnp.float32),
)


def kernel(outputs, index, confidence):
    partials = _sc_gather_dot(outputs, index, confidence)
    dense = _tc_dense(outputs)
    return _tc_combine(dense, partials)[0, 0]
